# baseline (speedup 1.0000x reference)
import functools

import jax
import jax.numpy as jnp
from jax import lax
from jax.experimental import pallas as pl
from jax.experimental.pallas import tpu as pltpu
from jax.experimental.pallas import tpu_sc as plsc

BATCH = 1024
HIDDEN = 1024
MEM_VEC = 128
BIN_LEN = 7

_NC = 1
_NS = 16
_NW = _NC * _NS
_B_PER_W = BATCH // _NW


def _tc_body(x_ref, w2_ref, b2_ref, w_ref, idx_ref):
    w = lax.dot_general(
        x_ref[...], w2_ref[...],
        dimension_numbers=(((1,), (1,)), ((), ())),
        preferred_element_type=jnp.float32,
    )
    w = jnp.maximum(w + b2_ref[...], 0.0)
    w_ref[...] = w
    bits = (w > 0.5).astype(jnp.float32)
    col = lax.broadcasted_iota(jnp.int32, (1, BIN_LEN), 1)
    powers = lax.shift_left(1, BIN_LEN - 1 - col).astype(jnp.float32)
    idx_row = lax.dot_general(
        powers, bits,
        dimension_numbers=(((1,), (1,)), ((), ())),
        preferred_element_type=jnp.float32,
    )
    idx_ref[...] = jnp.reshape(idx_row.astype(jnp.int32), (_TC_BLK,))


_TC_GRID = 2
_TC_BLK = BATCH // _TC_GRID

_tc_call = pl.pallas_call(
    _tc_body,
    grid=(_TC_GRID,),
    in_specs=[
        pl.BlockSpec((_TC_BLK, HIDDEN), lambda i: (i, 0)),
        pl.BlockSpec((BIN_LEN, HIDDEN), lambda i: (0, 0)),
        pl.BlockSpec((1, BIN_LEN), lambda i: (0, 0)),
    ],
    out_specs=[
        pl.BlockSpec((_TC_BLK, BIN_LEN), lambda i: (i, 0)),
        pl.BlockSpec((_TC_BLK,), lambda i: (i,)),
    ],
    out_shape=[
        jax.ShapeDtypeStruct((BATCH, BIN_LEN), jnp.float32),
        jax.ShapeDtypeStruct((BATCH,), jnp.int32),
    ],
)


@functools.cache
def _make_sc_gather():
    mesh = plsc.VectorSubcoreMesh(
        core_axis_name="c", subcore_axis_name="s",
        num_cores=_NC, num_subcores=_NS,
    )

    @functools.partial(
        pl.kernel,
        mesh=mesh,
        out_type=jax.ShapeDtypeStruct((BATCH, MEM_VEC), jnp.float32),
        scratch_types=[
            pltpu.VMEM((_B_PER_W,), jnp.int32),
            pltpu.VMEM((_B_PER_W, MEM_VEC), jnp.float32),
            pltpu.SemaphoreType.DMA,
            pltpu.SemaphoreType.DMA,
            pltpu.SemaphoreType.DMA,
            pltpu.SemaphoreType.DMA,
            pltpu.SemaphoreType.DMA,
            pltpu.SemaphoreType.DMA,
            pltpu.SemaphoreType.DMA,
            pltpu.SemaphoreType.DMA,
            pltpu.SemaphoreType.DMA,
            pltpu.SemaphoreType.DMA,
            pltpu.SemaphoreType.DMA,
            pltpu.SemaphoreType.DMA,
        ],
    )
    def _sc_gather(idx_hbm, table_hbm, out_hbm, idx_v, rows_v, *sems):
        isem, gsem, wsem = sems[:4], sems[4:8], sems[8:]
        wid = lax.axis_index("s") * _NC + lax.axis_index("c")
        base = wid * _B_PER_W
        q = _B_PER_W // 4
        iloads = []
        for k in range(4):
            iloads.append(pltpu.async_copy(
                idx_hbm.at[pl.ds(base + k * q, q)],
                idx_v.at[pl.ds(k * q, q)], isem[k]
            ))
        gathers = []
        for k in range(4):
            iloads[k].wait()
            gathers.append(pltpu.async_copy(
                table_hbm.at[idx_v.at[pl.ds(k * q, q)]],
                rows_v.at[pl.ds(k * q, q)], gsem[k]
            ))
        writes = []
        for k in range(4):
            gathers[k].wait()
            writes.append(pltpu.async_copy(
                rows_v.at[pl.ds(k * q, q)],
                out_hbm.at[pl.ds(base + k * q, q)], wsem[k]
            ))
        for wcp in writes:
            wcp.wait()

    return _sc_gather


def kernel(x, previous_state, W2, b2, memory):
    w, idx = _tc_call(x, W2, b2.reshape(1, BIN_LEN))
    memory_read = _make_sc_gather()(idx, memory)
    return memory_read, w


# speedup vs baseline: 1.0055x; 1.0055x over previous
import functools

import jax
import jax.numpy as jnp
from jax import lax
from jax.experimental import pallas as pl
from jax.experimental.pallas import tpu as pltpu
from jax.experimental.pallas import tpu_sc as plsc

BATCH = 1024
HIDDEN = 1024
MEM_VEC = 128
BIN_LEN = 7

_NC = 1
_NS = 16
_NW = _NC * _NS
_B_PER_W = BATCH // _NW


def _tc_body(x_ref, w2_ref, b2_ref, w_ref, idx_ref):
    w = lax.dot_general(
        x_ref[...], w2_ref[...],
        dimension_numbers=(((1,), (1,)), ((), ())),
        preferred_element_type=jnp.float32,
    )
    w = jnp.maximum(w + b2_ref[...], 0.0)
    w_ref[...] = w
    bits = (w > 0.5).astype(jnp.float32)
    col = lax.broadcasted_iota(jnp.int32, (1, BIN_LEN), 1)
    powers = lax.shift_left(1, BIN_LEN - 1 - col).astype(jnp.float32)
    idx_row = lax.dot_general(
        powers, bits,
        dimension_numbers=(((1,), (1,)), ((), ())),
        preferred_element_type=jnp.float32,
    )
    idx_ref[...] = jnp.reshape(idx_row.astype(jnp.int32), (_TC_BLK,))


_TC_GRID = 2
_TC_BLK = BATCH // _TC_GRID

_tc_call = pl.pallas_call(
    _tc_body,
    grid=(_TC_GRID,),
    in_specs=[
        pl.BlockSpec((_TC_BLK, HIDDEN), lambda i: (i, 0)),
        pl.BlockSpec((BIN_LEN, HIDDEN), lambda i: (0, 0)),
        pl.BlockSpec((1, BIN_LEN), lambda i: (0, 0)),
    ],
    out_specs=[
        pl.BlockSpec((_TC_BLK, BIN_LEN), lambda i: (i, 0)),
        pl.BlockSpec((_TC_BLK,), lambda i: (i,)),
    ],
    out_shape=[
        jax.ShapeDtypeStruct((BATCH, BIN_LEN), jnp.float32),
        jax.ShapeDtypeStruct((BATCH,), jnp.int32),
    ],
)


@functools.cache
def _make_sc_gather():
    mesh = plsc.VectorSubcoreMesh(
        core_axis_name="c", subcore_axis_name="s",
        num_cores=_NC, num_subcores=_NS,
    )

    @functools.partial(
        pl.kernel,
        mesh=mesh,
        out_type=jax.ShapeDtypeStruct((BATCH, MEM_VEC), jnp.float32),
        scratch_types=[
            pltpu.VMEM((_B_PER_W,), jnp.int32),
            pltpu.VMEM((_B_PER_W, MEM_VEC), jnp.float32),
            pltpu.SemaphoreType.DMA,
            pltpu.SemaphoreType.DMA,
            pltpu.SemaphoreType.DMA,
            pltpu.SemaphoreType.DMA,
            pltpu.SemaphoreType.DMA,
            pltpu.SemaphoreType.DMA,
            pltpu.SemaphoreType.DMA,
            pltpu.SemaphoreType.DMA,
        ],
    )
    def _sc_gather(idx_hbm, table_hbm, out_hbm, idx_v, rows_v, *sems):
        gsem, wsem = sems[:4], sems[4:]
        wid = lax.axis_index("s") * _NC + lax.axis_index("c")
        base = wid * _B_PER_W
        q = _B_PER_W // 4
        pltpu.sync_copy(idx_hbm.at[pl.ds(base, _B_PER_W)], idx_v)
        gathers = []
        for k in range(4):
            gathers.append(pltpu.async_copy(
                table_hbm.at[idx_v.at[pl.ds(k * q, q)]],
                rows_v.at[pl.ds(k * q, q)], gsem[k]
            ))
        writes = []
        for k in range(4):
            gathers[k].wait()
            writes.append(pltpu.async_copy(
                rows_v.at[pl.ds(k * q, q)],
                out_hbm.at[pl.ds(base + k * q, q)], wsem[k]
            ))
        for wcp in writes:
            wcp.wait()

    return _sc_gather


def kernel(x, previous_state, W2, b2, memory):
    w, idx = _tc_call(x, W2, b2.reshape(1, BIN_LEN))
    memory_read = _make_sc_gather()(idx, memory)
    return memory_read, w
